# baseline (speedup 1.0000x reference)
import functools

import jax
import jax.numpy as jnp
from jax import lax
from jax.experimental import pallas as pl
from jax.experimental.pallas import tpu as pltpu
from jax.experimental.pallas import tpu_sc as plsc

_F32 = jnp.float32
HID = 256


def _round_up(x, m):
    return (x + m - 1) // m * m


def _build_layout(src, dst, n_seg_pad, R, B):
    E = dst.shape[0]
    K = n_seg_pad // R
    E_pad = _round_up(E + K * B, 4096)
    G = E_pad // B

    order = jnp.argsort(dst).astype(jnp.int32)
    dst_s = jnp.take(dst, order)
    blk = dst_s // R
    cnt = jnp.zeros((K,), jnp.int32).at[blk].add(1)
    pk = jnp.maximum(((cnt + B - 1) // B) * B, B)
    zero1 = jnp.zeros((1,), jnp.int32)
    pstart = jnp.concatenate([zero1, jnp.cumsum(pk)])[:K]
    first = jnp.concatenate([zero1, jnp.cumsum(cnt)])[:K]
    rank = jnp.arange(E, dtype=jnp.int32) - jnp.take(first, blk)
    pos = jnp.take(pstart, blk) + rank

    bstart = pstart // B
    sp = jnp.clip(
        jnp.searchsorted(bstart, jnp.arange(G, dtype=jnp.int32), side="right")
        .astype(jnp.int32) - 1, 0, K - 1)

    valid = jnp.zeros((E_pad, 1), _F32).at[pos, 0].set(1.0)
    dloc = jnp.zeros((E_pad, 1), jnp.int32).at[pos, 0].set(dst_s - blk * R)
    take = jnp.zeros((E_pad,), jnp.int32).at[pos].set(order)
    pos_of_orig = jnp.zeros((E,), jnp.int32).at[order].set(pos)
    return dict(
        E_pad=E_pad, G=G, K=K, sp=sp, valid=valid, dloc=dloc, take=take,
        pos_of_orig=pos_of_orig,
        src=jnp.take(src, take), dst=jnp.take(dst, take))


@functools.cache
def _make_gather(V, D, B_total):
    C = 128
    NW = 32
    b_per_w = B_total // NW
    nch = b_per_w // C
    mesh = plsc.VectorSubcoreMesh(core_axis_name="c", subcore_axis_name="s")

    @functools.partial(
        pl.kernel,
        out_type=jax.ShapeDtypeStruct((B_total, D), _F32),
        mesh=mesh,
        scratch_types=[
            pltpu.VMEM((C,), jnp.int32),
            pltpu.VMEM((C, D), _F32),
            pltpu.SemaphoreType.DMA,
        ],
    )
    def gather(table_hbm, idx_hbm, out_hbm, idx_v, rows_v, sem):
        wid = lax.axis_index("s") * 2 + lax.axis_index("c")
        base = wid * b_per_w

        def body(j, carry):
            off = base + j * C
            pltpu.sync_copy(idx_hbm.at[pl.ds(off, C)], idx_v)
            pltpu.async_copy(table_hbm.at[idx_v], rows_v, sem).wait()
            pltpu.sync_copy(rows_v, out_hbm.at[pl.ds(off, C)])
            return carry

        lax.fori_loop(0, nch, body, 0)

    return gather


def _sc_gather(table, idx):
    V, D = table.shape
    (Bt,) = idx.shape
    return _make_gather(V, D, Bt)(table, idx)


def _mm_kernel(x_ref, w_ref, b_ref, o_ref):
    o_ref[...] = (jnp.dot(x_ref[...], w_ref[...], preferred_element_type=_F32)
                  + b_ref[...])


def _mm(x, w, b, br=512):
    n, k = x.shape
    m = w.shape[1]
    return pl.pallas_call(
        _mm_kernel,
        grid=(n // br,),
        in_specs=[
            pl.BlockSpec((br, k), lambda g: (g, 0)),
            pl.BlockSpec((k, m), lambda g: (0, 0)),
            pl.BlockSpec((1, m), lambda g: (0, 0)),
        ],
        out_specs=pl.BlockSpec((br, m), lambda g: (g, 0)),
        out_shape=jax.ShapeDtypeStruct((n, m), _F32),
    )(x, w, b.reshape(1, m))


def _mm_stats_kernel(x_ref, w_ref, b_ref, mask_ref, y_ref, st_ref):
    y = (jnp.dot(x_ref[...], w_ref[...], preferred_element_type=_F32)
         + b_ref[...])
    y_ref[...] = y

    @pl.when(pl.program_id(0) == 0)
    def _():
        st_ref[...] = jnp.zeros(st_ref.shape, _F32)

    yv = y * mask_ref[...]
    st_ref[0:1, :] += jnp.sum(yv, axis=0, keepdims=True)
    st_ref[1:2, :] += jnp.sum(y * yv, axis=0, keepdims=True)


def _mm_stats(x, w, b, mask, br=512):
    n, k = x.shape
    m = w.shape[1]
    return pl.pallas_call(
        _mm_stats_kernel,
        grid=(n // br,),
        in_specs=[
            pl.BlockSpec((br, k), lambda g: (g, 0)),
            pl.BlockSpec((k, m), lambda g: (0, 0)),
            pl.BlockSpec((1, m), lambda g: (0, 0)),
            pl.BlockSpec((br, 1), lambda g: (g, 0)),
        ],
        out_specs=[
            pl.BlockSpec((br, m), lambda g: (g, 0)),
            pl.BlockSpec((8, m), lambda g: (0, 0)),
        ],
        out_shape=[
            jax.ShapeDtypeStruct((n, m), _F32),
            jax.ShapeDtypeStruct((8, m), _F32),
        ],
    )(x, w, b.reshape(1, m), mask)


def _make_rbf_kernel(gamma):
    def k(d_ref, cen_ref, w_ref, b_ref, mask_ref, y_ref, st_ref):
        d = d_ref[...]
        f = jnp.exp(-gamma * (d - cen_ref[...]) ** 2)
        y = (jnp.dot(f, w_ref[...], preferred_element_type=_F32)
             + b_ref[...])
        y_ref[...] = y

        @pl.when(pl.program_id(0) == 0)
        def _():
            st_ref[...] = jnp.zeros(st_ref.shape, _F32)

        yv = y * mask_ref[...]
        st_ref[0:1, :] += jnp.sum(yv, axis=0, keepdims=True)
        st_ref[1:2, :] += jnp.sum(y * yv, axis=0, keepdims=True)

    return k


def _rbf_mm_stats(d, vmin, vmax, bins, w, b, mask, br=512):
    n = d.shape[0]
    m = w.shape[1]
    gamma = float((bins - 1) / (vmax - vmin))
    centers = jnp.linspace(vmin, vmax, bins, dtype=_F32).reshape(1, bins)
    return pl.pallas_call(
        _make_rbf_kernel(gamma),
        grid=(n // br,),
        in_specs=[
            pl.BlockSpec((br, 1), lambda g: (g, 0)),
            pl.BlockSpec((1, bins), lambda g: (0, 0)),
            pl.BlockSpec((bins, m), lambda g: (0, 0)),
            pl.BlockSpec((1, m), lambda g: (0, 0)),
            pl.BlockSpec((br, 1), lambda g: (g, 0)),
        ],
        out_specs=[
            pl.BlockSpec((br, m), lambda g: (g, 0)),
            pl.BlockSpec((8, m), lambda g: (0, 0)),
        ],
        out_shape=[
            jax.ShapeDtypeStruct((n, m), _F32),
            jax.ShapeDtypeStruct((8, m), _F32),
        ],
    )(d, centers, w, b.reshape(1, m), mask)


def _make_edge_kernel(R, B):
    def k(sp_ref, gs_ref, ds_ref, y_ref, w_ref, b_ref, dloc_ref, valid_ref,
          m_ref, nd_ref, st_ref):
        g = pl.program_id(0)
        a = gs_ref[:, :HID]
        c = gs_ref[:, HID:]
        eg = (jnp.dot(y_ref[...], w_ref[...], preferred_element_type=_F32)
              + b_ref[...])
        m = a + ds_ref[...] + eg
        m_ref[...] = m
        valid = valid_ref[...]
        sig = jax.nn.sigmoid(m) * valid
        contrib = jnp.concatenate([sig * c, sig], axis=1)
        onehot = (dloc_ref[...]
                  == lax.broadcasted_iota(jnp.int32, (B, R), 1)).astype(_F32)

        blk = sp_ref[g]
        prev = sp_ref[jnp.maximum(g - 1, 0)]

        @pl.when(jnp.logical_or(g == 0, blk != prev))
        def _():
            nd_ref[...] = jnp.zeros(nd_ref.shape, _F32)

        nd_ref[...] += lax.dot_general(
            onehot, contrib, (((0,), (0,)), ((), ())),
            preferred_element_type=_F32)

        @pl.when(g == 0)
        def _():
            st_ref[...] = jnp.zeros(st_ref.shape, _F32)

        mv = m * valid
        st_ref[0:1, :] += jnp.sum(mv, axis=0, keepdims=True)
        st_ref[1:2, :] += jnp.sum(m * mv, axis=0, keepdims=True)

    return k


def _edge_stage(lay, gs, ds, y, w, b, R, B, n_seg_pad):
    E_pad = gs.shape[0]
    G = E_pad // B
    grid_spec = pltpu.PrefetchScalarGridSpec(
        num_scalar_prefetch=1,
        grid=(G,),
        in_specs=[
            pl.BlockSpec((B, 2 * HID), lambda g, sp: (g, 0)),
            pl.BlockSpec((B, HID), lambda g, sp: (g, 0)),
            pl.BlockSpec((B, HID), lambda g, sp: (g, 0)),
            pl.BlockSpec((HID, HID), lambda g, sp: (0, 0)),
            pl.BlockSpec((1, HID), lambda g, sp: (0, 0)),
            pl.BlockSpec((B, 1), lambda g, sp: (g, 0)),
            pl.BlockSpec((B, 1), lambda g, sp: (g, 0)),
        ],
        out_specs=[
            pl.BlockSpec((B, HID), lambda g, sp: (g, 0)),
            pl.BlockSpec((R, 2 * HID), lambda g, sp: (sp[g], 0)),
            pl.BlockSpec((8, HID), lambda g, sp: (0, 0)),
        ],
    )
    return pl.pallas_call(
        _make_edge_kernel(R, B),
        grid_spec=grid_spec,
        out_shape=[
            jax.ShapeDtypeStruct((E_pad, HID), _F32),
            jax.ShapeDtypeStruct((n_seg_pad, 2 * HID), _F32),
            jax.ShapeDtypeStruct((8, HID), _F32),
        ],
    )(lay["sp"], gs, ds, y, w, b.reshape(1, HID), lay["dloc"], lay["valid"])


def _node_kernel(x_ref, w_ref, b_ref, nd_ref, mask_ref, t_ref, st_ref):
    nd = nd_ref[...]
    h = nd[:, :HID] / (nd[:, HID:] + 1e-6)
    t = (jnp.dot(x_ref[...], w_ref[...], preferred_element_type=_F32)
         + b_ref[...] + h)
    t_ref[...] = t

    @pl.when(pl.program_id(0) == 0)
    def _():
        st_ref[...] = jnp.zeros(st_ref.shape, _F32)

    tv = t * mask_ref[...]
    st_ref[0:1, :] += jnp.sum(tv, axis=0, keepdims=True)
    st_ref[1:2, :] += jnp.sum(t * tv, axis=0, keepdims=True)


def _node_stage(x, w, b, nd, mask, br=512):
    n = x.shape[0]
    return pl.pallas_call(
        _node_kernel,
        grid=(n // br,),
        in_specs=[
            pl.BlockSpec((br, HID), lambda g: (g, 0)),
            pl.BlockSpec((HID, HID), lambda g: (0, 0)),
            pl.BlockSpec((1, HID), lambda g: (0, 0)),
            pl.BlockSpec((br, 2 * HID), lambda g: (g, 0)),
            pl.BlockSpec((br, 1), lambda g: (g, 0)),
        ],
        out_specs=[
            pl.BlockSpec((br, HID), lambda g: (g, 0)),
            pl.BlockSpec((8, HID), lambda g: (0, 0)),
        ],
        out_shape=[
            jax.ShapeDtypeStruct((n, HID), _F32),
            jax.ShapeDtypeStruct((8, HID), _F32),
        ],
    )(x, w, b.reshape(1, HID), nd, mask)


def _apply_res_kernel(y_ref, r_ref, sc_ref, sh_ref, o_ref):
    yb = y_ref[...] * sc_ref[...] + sh_ref[...]
    o_ref[...] = r_ref[...] + yb * jax.nn.sigmoid(yb)


def _apply_kernel(y_ref, sc_ref, sh_ref, o_ref):
    yb = y_ref[...] * sc_ref[...] + sh_ref[...]
    o_ref[...] = yb * jax.nn.sigmoid(yb)


def _apply(y, sc, sh, res=None, br=512):
    n, m = y.shape
    row = pl.BlockSpec((br, m), lambda g: (g, 0))
    one = pl.BlockSpec((1, m), lambda g: (0, 0))
    if res is None:
        return pl.pallas_call(
            _apply_kernel, grid=(n // br,),
            in_specs=[row, one, one], out_specs=row,
            out_shape=jax.ShapeDtypeStruct((n, m), _F32),
        )(y, sc.reshape(1, m), sh.reshape(1, m))
    return pl.pallas_call(
        _apply_res_kernel, grid=(n // br,),
        in_specs=[row, row, one, one], out_specs=row,
        out_shape=jax.ShapeDtypeStruct((n, m), _F32),
    )(y, res, sc.reshape(1, m), sh.reshape(1, m))


def _colsum_kernel(x_ref, mask_ref, st_ref):
    @pl.when(pl.program_id(0) == 0)
    def _():
        st_ref[...] = jnp.zeros(st_ref.shape, _F32)

    st_ref[0:1, :] += jnp.sum(x_ref[...] * mask_ref[...], axis=0,
                              keepdims=True)


def _colsum(x, mask, br=512):
    n, m = x.shape
    return pl.pallas_call(
        _colsum_kernel,
        grid=(n // br,),
        in_specs=[
            pl.BlockSpec((br, m), lambda g: (g, 0)),
            pl.BlockSpec((br, 1), lambda g: (g, 0)),
        ],
        out_specs=pl.BlockSpec((8, m), lambda g: (0, 0)),
        out_shape=jax.ShapeDtypeStruct((8, m), _F32),
    )(x, mask)


def _bn_affine(st, count, gamma, beta):
    s = st[0]
    ss = st[1]
    mu = s / count
    var = ss / count - mu * mu
    inv = gamma * lax.rsqrt(var + 1e-5)
    return inv, beta - mu * inv


def _mlp(p, x, mask, count):
    y, st = _mm_stats(x, p["w"], p["b"], mask)
    sc, sh = _bn_affine(st, count, p["g"], p["be"])
    return _apply(y, sc, sh)


def _eggc(p, lay, R, B, n_seg_pad, x, y, x_mask, x_count, y_count):
    wg = jnp.concatenate([p["src_gate_w"], p["dst_update_w"]], axis=1)
    bg = jnp.concatenate([p["src_gate_b"], p["dst_update_b"]])
    gt = _mm(x, wg, bg)
    dt = _mm(x, p["dst_gate_w"], p["dst_gate_b"])
    gs = _sc_gather(gt, lay["src"])
    ds = _sc_gather(dt, lay["dst"])
    m, nd, mst = _edge_stage(lay, gs, ds, y, p["edge_gate_w"],
                             p["edge_gate_b"], R, B, n_seg_pad)
    t, tst = _node_stage(x, p["src_update_w"], p["src_update_b"], nd, x_mask)
    tsc, tsh = _bn_affine(tst, x_count, p["bn_nodes_g"], p["bn_nodes_b"])
    msc, msh = _bn_affine(mst, y_count, p["bn_edges_g"], p["bn_edges_b"])
    x_out = _apply(t, tsc, tsh, res=x)
    y_out = _apply(m, msc, msh, res=y)
    return x_out, y_out


def kernel(atom_features, r, angle_h, edge_index, lg_edge_index, params):
    N = atom_features.shape[0]
    E = r.shape[0]
    T = angle_h.shape[0]
    R_G, B_G = 128, 128
    R_L, B_L = 128, 128

    N_pad = _round_up(N, 512)
    src = edge_index[0].astype(jnp.int32)
    dst = edge_index[1].astype(jnp.int32)
    gl = _build_layout(src, dst, N_pad, R_G, B_G)
    E_pad = gl["E_pad"]

    lsrc = jnp.take(gl["pos_of_orig"], lg_edge_index[0].astype(jnp.int32))
    ldst = jnp.take(gl["pos_of_orig"], lg_edge_index[1].astype(jnp.int32))
    ll = _build_layout(lsrc, ldst, E_pad, R_L, B_L)
    T_pad = ll["E_pad"]

    node_mask = (jnp.arange(N_pad) < N).astype(_F32).reshape(N_pad, 1)
    edge_mask = gl["valid"]
    ang_mask = ll["valid"]

    x0 = jnp.zeros((N_pad, atom_features.shape[1]), _F32).at[:N].set(
        atom_features)
    x = _mlp(params["atom_emb"], x0, node_mask, float(N))

    d_bond = jnp.sqrt(jnp.sum(r * r, axis=1))
    d_pad = jnp.take(d_bond, gl["take"]).reshape(E_pad, 1)
    p1 = params["edge_emb1"]
    y, st = _rbf_mm_stats(d_pad, 0.0, 8.0, 80, p1["w"], p1["b"], edge_mask)
    sc, sh = _bn_affine(st, float(E), p1["g"], p1["be"])
    y = _apply(y, sc, sh)
    y = _mlp(params["edge_emb2"], y, edge_mask, float(E))

    d_ang = jnp.take(angle_h, ll["take"]).reshape(T_pad, 1)
    p2 = params["angle_emb1"]
    z, st = _rbf_mm_stats(d_ang, -1.0, 1.0, 40, p2["w"], p2["b"], ang_mask)
    sc, sh = _bn_affine(st, float(T), p2["g"], p2["be"])
    z = _apply(z, sc, sh)
    z = _mlp(params["angle_emb2"], z, ang_mask, float(T))

    for lp in params["alignn"]:
        x, m = _eggc(lp["node"], gl, R_G, B_G, N_pad, x, y, node_mask,
                     float(N), float(E))
        y, z = _eggc(lp["edge"], ll, R_L, B_L, E_pad, m, z, edge_mask,
                     float(E), float(T))
    for lp in params["gcn"]:
        x, y = _eggc(lp, gl, R_G, B_G, N_pad, x, y, node_mask,
                     float(N), float(E))

    st = _colsum(x, node_mask)
    h = st[0] / float(N)
    out = h @ params["fc_w"] + params["fc_b"]
    return jnp.squeeze(out)


# speedup vs baseline: 1.4208x; 1.4208x over previous
import functools

import jax
import jax.numpy as jnp
from jax import lax
from jax.experimental import pallas as pl
from jax.experimental.pallas import tpu as pltpu
from jax.experimental.pallas import tpu_sc as plsc

_F32 = jnp.float32
HID = 256


def _round_up(x, m):
    return (x + m - 1) // m * m


def _build_layout(src, dst, n_seg_pad, R, B):
    E = dst.shape[0]
    K = n_seg_pad // R
    E_pad = _round_up(E + K * B, 4096)
    G = E_pad // B

    order = jnp.argsort(dst).astype(jnp.int32)
    dst_s = jnp.take(dst, order)
    blk = dst_s // R
    cnt = jnp.zeros((K,), jnp.int32).at[blk].add(1)
    pk = jnp.maximum(((cnt + B - 1) // B) * B, B)
    zero1 = jnp.zeros((1,), jnp.int32)
    pstart = jnp.concatenate([zero1, jnp.cumsum(pk)])[:K]
    first = jnp.concatenate([zero1, jnp.cumsum(cnt)])[:K]
    rank = jnp.arange(E, dtype=jnp.int32) - jnp.take(first, blk)
    pos = jnp.take(pstart, blk) + rank

    bstart = pstart // B
    sp = jnp.clip(
        jnp.searchsorted(bstart, jnp.arange(G, dtype=jnp.int32), side="right")
        .astype(jnp.int32) - 1, 0, K - 1)

    valid = jnp.zeros((E_pad, 1), _F32).at[pos, 0].set(1.0)
    dloc = jnp.zeros((E_pad, 1), jnp.int32).at[pos, 0].set(dst_s - blk * R)
    take = jnp.zeros((E_pad,), jnp.int32).at[pos].set(order)
    pos_of_orig = jnp.zeros((E,), jnp.int32).at[order].set(pos)
    return dict(
        E_pad=E_pad, G=G, K=K, sp=sp, valid=valid, dloc=dloc, take=take,
        pos_of_orig=pos_of_orig,
        src=jnp.take(src, take), dst=jnp.take(dst, take))


@functools.cache
def _make_gather(V, D, B_total):
    C = 64
    NW = 32
    b_per_w = B_total // NW
    nch = b_per_w // C
    assert b_per_w % C == 0 and nch % 2 == 0
    mesh = plsc.VectorSubcoreMesh(core_axis_name="c", subcore_axis_name="s")

    @functools.partial(
        pl.kernel,
        out_type=jax.ShapeDtypeStruct((B_total, D), _F32),
        mesh=mesh,
        scratch_types=[
            pltpu.VMEM((C,), jnp.int32),
            pltpu.VMEM((C,), jnp.int32),
            pltpu.VMEM((C, D), _F32),
            pltpu.VMEM((C, D), _F32),
            pltpu.SemaphoreType.DMA,
            pltpu.SemaphoreType.DMA,
        ],
    )
    def gather(table_hbm, idx_hbm, out_hbm, idx0, idx1, rows0, rows1,
               sem0, sem1):
        wid = lax.axis_index("s") * 2 + lax.axis_index("c")
        base = wid * b_per_w
        idx_v = (idx0, idx1)
        rows_v = (rows0, rows1)
        sems = (sem0, sem1)

        pltpu.sync_copy(idx_hbm.at[pl.ds(base, C)], idx0)
        pltpu.async_copy(table_hbm.at[idx0], rows0, sem0)

        def body(k, carry):
            for b in range(2):
                j = 2 * k + b
                nb = 1 - b

                @pl.when(j + 1 < nch)
                def _(j=j, nb=nb):
                    off = base + (j + 1) * C
                    pltpu.sync_copy(idx_hbm.at[pl.ds(off, C)], idx_v[nb])
                    pltpu.async_copy(table_hbm.at[idx_v[nb]], rows_v[nb],
                                     sems[nb])

                pltpu.make_async_copy(table_hbm.at[idx_v[b]], rows_v[b],
                                      sems[b]).wait()
                pltpu.sync_copy(rows_v[b], out_hbm.at[pl.ds(base + j * C, C)])
            return carry

        lax.fori_loop(0, nch // 2, body, 0)

    return gather


def _sc_gather(table, idx):
    V, D = table.shape
    (Bt,) = idx.shape
    return _make_gather(V, D, Bt)(table, idx)


def _mm_kernel(x_ref, w_ref, b_ref, o_ref):
    o_ref[...] = (jnp.dot(x_ref[...], w_ref[...], preferred_element_type=_F32)
                  + b_ref[...])


def _mm(x, w, b, br=512):
    n, k = x.shape
    m = w.shape[1]
    return pl.pallas_call(
        _mm_kernel,
        grid=(n // br,),
        in_specs=[
            pl.BlockSpec((br, k), lambda g: (g, 0)),
            pl.BlockSpec((k, m), lambda g: (0, 0)),
            pl.BlockSpec((1, m), lambda g: (0, 0)),
        ],
        out_specs=pl.BlockSpec((br, m), lambda g: (g, 0)),
        out_shape=jax.ShapeDtypeStruct((n, m), _F32),
    )(x, w, b.reshape(1, m))


def _mm_stats_kernel(x_ref, w_ref, b_ref, mask_ref, y_ref, st_ref):
    y = (jnp.dot(x_ref[...], w_ref[...], preferred_element_type=_F32)
         + b_ref[...])
    y_ref[...] = y

    @pl.when(pl.program_id(0) == 0)
    def _():
        st_ref[...] = jnp.zeros(st_ref.shape, _F32)

    yv = y * mask_ref[...]
    st_ref[0:1, :] += jnp.sum(yv, axis=0, keepdims=True)
    st_ref[1:2, :] += jnp.sum(y * yv, axis=0, keepdims=True)


def _mm_stats(x, w, b, mask, br=512):
    n, k = x.shape
    m = w.shape[1]
    return pl.pallas_call(
        _mm_stats_kernel,
        grid=(n // br,),
        in_specs=[
            pl.BlockSpec((br, k), lambda g: (g, 0)),
            pl.BlockSpec((k, m), lambda g: (0, 0)),
            pl.BlockSpec((1, m), lambda g: (0, 0)),
            pl.BlockSpec((br, 1), lambda g: (g, 0)),
        ],
        out_specs=[
            pl.BlockSpec((br, m), lambda g: (g, 0)),
            pl.BlockSpec((8, m), lambda g: (0, 0)),
        ],
        out_shape=[
            jax.ShapeDtypeStruct((n, m), _F32),
            jax.ShapeDtypeStruct((8, m), _F32),
        ],
    )(x, w, b.reshape(1, m), mask)


def _make_rbf_kernel(gamma):
    def k(d_ref, cen_ref, w_ref, b_ref, mask_ref, y_ref, st_ref):
        d = d_ref[...]
        f = jnp.exp(-gamma * (d - cen_ref[...]) ** 2)
        y = (jnp.dot(f, w_ref[...], preferred_element_type=_F32)
             + b_ref[...])
        y_ref[...] = y

        @pl.when(pl.program_id(0) == 0)
        def _():
            st_ref[...] = jnp.zeros(st_ref.shape, _F32)

        yv = y * mask_ref[...]
        st_ref[0:1, :] += jnp.sum(yv, axis=0, keepdims=True)
        st_ref[1:2, :] += jnp.sum(y * yv, axis=0, keepdims=True)

    return k


def _rbf_mm_stats(d, vmin, vmax, bins, w, b, mask, br=512):
    n = d.shape[0]
    m = w.shape[1]
    gamma = float((bins - 1) / (vmax - vmin))
    centers = jnp.linspace(vmin, vmax, bins, dtype=_F32).reshape(1, bins)
    return pl.pallas_call(
        _make_rbf_kernel(gamma),
        grid=(n // br,),
        in_specs=[
            pl.BlockSpec((br, 1), lambda g: (g, 0)),
            pl.BlockSpec((1, bins), lambda g: (0, 0)),
            pl.BlockSpec((bins, m), lambda g: (0, 0)),
            pl.BlockSpec((1, m), lambda g: (0, 0)),
            pl.BlockSpec((br, 1), lambda g: (g, 0)),
        ],
        out_specs=[
            pl.BlockSpec((br, m), lambda g: (g, 0)),
            pl.BlockSpec((8, m), lambda g: (0, 0)),
        ],
        out_shape=[
            jax.ShapeDtypeStruct((n, m), _F32),
            jax.ShapeDtypeStruct((8, m), _F32),
        ],
    )(d, centers, w, b.reshape(1, m), mask)


def _make_edge_kernel(R, B):
    def k(sp_ref, gs_ref, dt_ref, y_ref, w_ref, b_ref, dloc_ref, valid_ref,
          m_ref, nd_ref, st_ref):
        g = pl.program_id(0)
        a = gs_ref[:, :HID]
        c = gs_ref[:, HID:]
        eg = (jnp.dot(y_ref[...], w_ref[...], preferred_element_type=_F32)
              + b_ref[...])
        onehot = (dloc_ref[...]
                  == lax.broadcasted_iota(jnp.int32, (B, R), 1)).astype(_F32)
        b_rows = jnp.dot(onehot, dt_ref[...], preferred_element_type=_F32)
        m = a + b_rows + eg
        m_ref[...] = m
        valid = valid_ref[...]
        sig = jax.nn.sigmoid(m) * valid
        contrib = jnp.concatenate([sig * c, sig], axis=1)

        blk = sp_ref[g]
        prev = sp_ref[jnp.maximum(g - 1, 0)]

        @pl.when(jnp.logical_or(g == 0, blk != prev))
        def _():
            nd_ref[...] = jnp.zeros(nd_ref.shape, _F32)

        nd_ref[...] += lax.dot_general(
            onehot, contrib, (((0,), (0,)), ((), ())),
            preferred_element_type=_F32)

        @pl.when(g == 0)
        def _():
            st_ref[...] = jnp.zeros(st_ref.shape, _F32)

        mv = m * valid
        st_ref[0:1, :] += jnp.sum(mv, axis=0, keepdims=True)
        st_ref[1:2, :] += jnp.sum(m * mv, axis=0, keepdims=True)

    return k


def _edge_stage(lay, gs, dt, y, w, b, R, B, n_seg_pad):
    E_pad = gs.shape[0]
    G = E_pad // B
    grid_spec = pltpu.PrefetchScalarGridSpec(
        num_scalar_prefetch=1,
        grid=(G,),
        in_specs=[
            pl.BlockSpec((B, 2 * HID), lambda g, sp: (g, 0)),
            pl.BlockSpec((R, HID), lambda g, sp: (sp[g], 0)),
            pl.BlockSpec((B, HID), lambda g, sp: (g, 0)),
            pl.BlockSpec((HID, HID), lambda g, sp: (0, 0)),
            pl.BlockSpec((1, HID), lambda g, sp: (0, 0)),
            pl.BlockSpec((B, 1), lambda g, sp: (g, 0)),
            pl.BlockSpec((B, 1), lambda g, sp: (g, 0)),
        ],
        out_specs=[
            pl.BlockSpec((B, HID), lambda g, sp: (g, 0)),
            pl.BlockSpec((R, 2 * HID), lambda g, sp: (sp[g], 0)),
            pl.BlockSpec((8, HID), lambda g, sp: (0, 0)),
        ],
    )
    return pl.pallas_call(
        _make_edge_kernel(R, B),
        grid_spec=grid_spec,
        out_shape=[
            jax.ShapeDtypeStruct((E_pad, HID), _F32),
            jax.ShapeDtypeStruct((n_seg_pad, 2 * HID), _F32),
            jax.ShapeDtypeStruct((8, HID), _F32),
        ],
    )(lay["sp"], gs, dt, y, w, b.reshape(1, HID), lay["dloc"], lay["valid"])


def _node_kernel(x_ref, w_ref, b_ref, nd_ref, mask_ref, t_ref, st_ref):
    nd = nd_ref[...]
    h = nd[:, :HID] / (nd[:, HID:] + 1e-6)
    t = (jnp.dot(x_ref[...], w_ref[...], preferred_element_type=_F32)
         + b_ref[...] + h)
    t_ref[...] = t

    @pl.when(pl.program_id(0) == 0)
    def _():
        st_ref[...] = jnp.zeros(st_ref.shape, _F32)

    tv = t * mask_ref[...]
    st_ref[0:1, :] += jnp.sum(tv, axis=0, keepdims=True)
    st_ref[1:2, :] += jnp.sum(t * tv, axis=0, keepdims=True)


def _node_stage(x, w, b, nd, mask, br=512):
    n = x.shape[0]
    return pl.pallas_call(
        _node_kernel,
        grid=(n // br,),
        in_specs=[
            pl.BlockSpec((br, HID), lambda g: (g, 0)),
            pl.BlockSpec((HID, HID), lambda g: (0, 0)),
            pl.BlockSpec((1, HID), lambda g: (0, 0)),
            pl.BlockSpec((br, 2 * HID), lambda g: (g, 0)),
            pl.BlockSpec((br, 1), lambda g: (g, 0)),
        ],
        out_specs=[
            pl.BlockSpec((br, HID), lambda g: (g, 0)),
            pl.BlockSpec((8, HID), lambda g: (0, 0)),
        ],
        out_shape=[
            jax.ShapeDtypeStruct((n, HID), _F32),
            jax.ShapeDtypeStruct((8, HID), _F32),
        ],
    )(x, w, b.reshape(1, HID), nd, mask)


def _apply_res_kernel(y_ref, r_ref, sc_ref, sh_ref, o_ref):
    yb = y_ref[...] * sc_ref[...] + sh_ref[...]
    o_ref[...] = r_ref[...] + yb * jax.nn.sigmoid(yb)


def _apply_kernel(y_ref, sc_ref, sh_ref, o_ref):
    yb = y_ref[...] * sc_ref[...] + sh_ref[...]
    o_ref[...] = yb * jax.nn.sigmoid(yb)


def _apply(y, sc, sh, res=None, br=512):
    n, m = y.shape
    row = pl.BlockSpec((br, m), lambda g: (g, 0))
    one = pl.BlockSpec((1, m), lambda g: (0, 0))
    if res is None:
        return pl.pallas_call(
            _apply_kernel, grid=(n // br,),
            in_specs=[row, one, one], out_specs=row,
            out_shape=jax.ShapeDtypeStruct((n, m), _F32),
        )(y, sc.reshape(1, m), sh.reshape(1, m))
    return pl.pallas_call(
        _apply_res_kernel, grid=(n // br,),
        in_specs=[row, row, one, one], out_specs=row,
        out_shape=jax.ShapeDtypeStruct((n, m), _F32),
    )(y, res, sc.reshape(1, m), sh.reshape(1, m))


def _colsum_kernel(x_ref, mask_ref, st_ref):
    @pl.when(pl.program_id(0) == 0)
    def _():
        st_ref[...] = jnp.zeros(st_ref.shape, _F32)

    st_ref[0:1, :] += jnp.sum(x_ref[...] * mask_ref[...], axis=0,
                              keepdims=True)


def _colsum(x, mask, br=512):
    n, m = x.shape
    return pl.pallas_call(
        _colsum_kernel,
        grid=(n // br,),
        in_specs=[
            pl.BlockSpec((br, m), lambda g: (g, 0)),
            pl.BlockSpec((br, 1), lambda g: (g, 0)),
        ],
        out_specs=pl.BlockSpec((8, m), lambda g: (0, 0)),
        out_shape=jax.ShapeDtypeStruct((8, m), _F32),
    )(x, mask)


def _bn_affine(st, count, gamma, beta):
    s = st[0]
    ss = st[1]
    mu = s / count
    var = ss / count - mu * mu
    inv = gamma * lax.rsqrt(var + 1e-5)
    return inv, beta - mu * inv


def _mlp(p, x, mask, count):
    y, st = _mm_stats(x, p["w"], p["b"], mask)
    sc, sh = _bn_affine(st, count, p["g"], p["be"])
    return _apply(y, sc, sh)


def _eggc(p, lay, R, B, n_seg_pad, x, y, x_mask, x_count, y_count):
    wg = jnp.concatenate([p["src_gate_w"], p["dst_update_w"]], axis=1)
    bg = jnp.concatenate([p["src_gate_b"], p["dst_update_b"]])
    gt = _mm(x, wg, bg)
    dt = _mm(x, p["dst_gate_w"], p["dst_gate_b"])
    gs = _sc_gather(gt, lay["src"])
    m, nd, mst = _edge_stage(lay, gs, dt, y, p["edge_gate_w"],
                             p["edge_gate_b"], R, B, n_seg_pad)
    t, tst = _node_stage(x, p["src_update_w"], p["src_update_b"], nd, x_mask)
    tsc, tsh = _bn_affine(tst, x_count, p["bn_nodes_g"], p["bn_nodes_b"])
    msc, msh = _bn_affine(mst, y_count, p["bn_edges_g"], p["bn_edges_b"])
    x_out = _apply(t, tsc, tsh, res=x)
    y_out = _apply(m, msc, msh, res=y)
    return x_out, y_out


def kernel(atom_features, r, angle_h, edge_index, lg_edge_index, params):
    N = atom_features.shape[0]
    E = r.shape[0]
    T = angle_h.shape[0]
    R_G, B_G = 128, 128
    R_L, B_L = 128, 128

    N_pad = _round_up(N, 512)
    src = edge_index[0].astype(jnp.int32)
    dst = edge_index[1].astype(jnp.int32)
    gl = _build_layout(src, dst, N_pad, R_G, B_G)
    E_pad = gl["E_pad"]

    lsrc = jnp.take(gl["pos_of_orig"], lg_edge_index[0].astype(jnp.int32))
    ldst = jnp.take(gl["pos_of_orig"], lg_edge_index[1].astype(jnp.int32))
    ll = _build_layout(lsrc, ldst, E_pad, R_L, B_L)
    T_pad = ll["E_pad"]

    node_mask = (jnp.arange(N_pad) < N).astype(_F32).reshape(N_pad, 1)
    edge_mask = gl["valid"]
    ang_mask = ll["valid"]

    x0 = jnp.zeros((N_pad, atom_features.shape[1]), _F32).at[:N].set(
        atom_features)
    x = _mlp(params["atom_emb"], x0, node_mask, float(N))

    d_bond = jnp.sqrt(jnp.sum(r * r, axis=1))
    d_pad = jnp.take(d_bond, gl["take"]).reshape(E_pad, 1)
    p1 = params["edge_emb1"]
    y, st = _rbf_mm_stats(d_pad, 0.0, 8.0, 80, p1["w"], p1["b"], edge_mask)
    sc, sh = _bn_affine(st, float(E), p1["g"], p1["be"])
    y = _apply(y, sc, sh)
    y = _mlp(params["edge_emb2"], y, edge_mask, float(E))

    d_ang = jnp.take(angle_h, ll["take"]).reshape(T_pad, 1)
    p2 = params["angle_emb1"]
    z, st = _rbf_mm_stats(d_ang, -1.0, 1.0, 40, p2["w"], p2["b"], ang_mask)
    sc, sh = _bn_affine(st, float(T), p2["g"], p2["be"])
    z = _apply(z, sc, sh)
    z = _mlp(params["angle_emb2"], z, ang_mask, float(T))

    for lp in params["alignn"]:
        x, m = _eggc(lp["node"], gl, R_G, B_G, N_pad, x, y, node_mask,
                     float(N), float(E))
        y, z = _eggc(lp["edge"], ll, R_L, B_L, E_pad, m, z, edge_mask,
                     float(E), float(T))
    for lp in params["gcn"]:
        x, y = _eggc(lp, gl, R_G, B_G, N_pad, x, y, node_mask,
                     float(N), float(E))

    st = _colsum(x, node_mask)
    h = st[0] / float(N)
    out = h @ params["fc_w"] + params["fc_b"]
    return jnp.squeeze(out)


# speedup vs baseline: 1.5842x; 1.1150x over previous
import functools

import jax
import jax.numpy as jnp
from jax import lax
from jax.experimental import pallas as pl
from jax.experimental.pallas import tpu as pltpu
from jax.experimental.pallas import tpu_sc as plsc

_F32 = jnp.float32
HID = 256


def _round_up(x, m):
    return (x + m - 1) // m * m


def _build_layout(src, dst, n_seg_pad, R, B):
    E = dst.shape[0]
    K = n_seg_pad // R
    E_pad = _round_up(E + K * B, 8192)
    G = E_pad // B

    order = jnp.argsort(dst).astype(jnp.int32)
    dst_s = jnp.take(dst, order)
    blk = dst_s // R
    cnt = jnp.zeros((K,), jnp.int32).at[blk].add(1)
    pk = jnp.maximum(((cnt + B - 1) // B) * B, B)
    zero1 = jnp.zeros((1,), jnp.int32)
    pstart = jnp.concatenate([zero1, jnp.cumsum(pk)])[:K]
    first = jnp.concatenate([zero1, jnp.cumsum(cnt)])[:K]
    rank = jnp.arange(E, dtype=jnp.int32) - jnp.take(first, blk)
    pos = jnp.take(pstart, blk) + rank

    bstart = pstart // B
    sp = jnp.clip(
        jnp.searchsorted(bstart, jnp.arange(G, dtype=jnp.int32), side="right")
        .astype(jnp.int32) - 1, 0, K - 1)

    valid = jnp.zeros((E_pad, 1), _F32).at[pos, 0].set(1.0)
    dloc = jnp.zeros((E_pad, 1), jnp.int32).at[pos, 0].set(dst_s - blk * R)
    take = jnp.zeros((E_pad,), jnp.int32).at[pos].set(order)
    pos_of_orig = jnp.zeros((E,), jnp.int32).at[order].set(pos)
    return dict(
        E_pad=E_pad, G=G, K=K, sp=sp, valid=valid, dloc=dloc, take=take,
        pos_of_orig=pos_of_orig,
        src=jnp.take(src, take), dst=jnp.take(dst, take))


@functools.cache
def _make_gather(V, D, B_total):
    NW = 32
    b_per_w = B_total // NW
    C = max(c for c in (128, 112, 96, 80, 64, 48, 32, 16, 8)
            if b_per_w % c == 0 and (b_per_w // c) % 2 == 0
            and 2 * c * D * 4 <= 420 * 1024)
    nch = b_per_w // C
    mesh = plsc.VectorSubcoreMesh(core_axis_name="c", subcore_axis_name="s")

    @functools.partial(
        pl.kernel,
        out_type=jax.ShapeDtypeStruct((B_total, D), _F32),
        mesh=mesh,
        scratch_types=[
            pltpu.VMEM((C,), jnp.int32),
            pltpu.VMEM((C,), jnp.int32),
            pltpu.VMEM((C, D), _F32),
            pltpu.VMEM((C, D), _F32),
            pltpu.SemaphoreType.DMA,
            pltpu.SemaphoreType.DMA,
        ],
    )
    def gather(table_hbm, idx_hbm, out_hbm, idx0, idx1, rows0, rows1,
               sem0, sem1):
        wid = lax.axis_index("s") * 2 + lax.axis_index("c")
        base = wid * b_per_w
        idx_v = (idx0, idx1)
        rows_v = (rows0, rows1)
        sems = (sem0, sem1)

        pltpu.sync_copy(idx_hbm.at[pl.ds(base, C)], idx0)
        pltpu.async_copy(table_hbm.at[idx0], rows0, sem0)

        def body(k, carry):
            for b in range(2):
                j = 2 * k + b
                nb = 1 - b

                @pl.when(j + 1 < nch)
                def _(j=j, nb=nb):
                    off = base + (j + 1) * C
                    pltpu.sync_copy(idx_hbm.at[pl.ds(off, C)], idx_v[nb])
                    pltpu.async_copy(table_hbm.at[idx_v[nb]], rows_v[nb],
                                     sems[nb])

                pltpu.make_async_copy(table_hbm.at[idx_v[b]], rows_v[b],
                                      sems[b]).wait()
                pltpu.sync_copy(rows_v[b], out_hbm.at[pl.ds(base + j * C, C)])
            return carry

        lax.fori_loop(0, nch // 2, body, 0)

    return gather


def _sc_gather(table, idx):
    V, D = table.shape
    (Bt,) = idx.shape
    return _make_gather(V, D, Bt)(table, idx)


def _mm_kernel(x_ref, w_ref, b_ref, o_ref):
    o_ref[...] = (jnp.dot(x_ref[...], w_ref[...], preferred_element_type=_F32)
                  + b_ref[...])


def _mm(x, w, b, br=512):
    n, k = x.shape
    m = w.shape[1]
    return pl.pallas_call(
        _mm_kernel,
        grid=(n // br,),
        in_specs=[
            pl.BlockSpec((br, k), lambda g: (g, 0)),
            pl.BlockSpec((k, m), lambda g: (0, 0)),
            pl.BlockSpec((1, m), lambda g: (0, 0)),
        ],
        out_specs=pl.BlockSpec((br, m), lambda g: (g, 0)),
        out_shape=jax.ShapeDtypeStruct((n, m), _F32),
    )(x, w, b.reshape(1, m))


def _mm_stats_kernel(x_ref, w_ref, b_ref, mask_ref, y_ref, st_ref):
    y = (jnp.dot(x_ref[...], w_ref[...], preferred_element_type=_F32)
         + b_ref[...])
    y_ref[...] = y

    @pl.when(pl.program_id(0) == 0)
    def _():
        st_ref[...] = jnp.zeros(st_ref.shape, _F32)

    yv = y * mask_ref[...]
    st_ref[0:1, :] += jnp.sum(yv, axis=0, keepdims=True)
    st_ref[1:2, :] += jnp.sum(y * yv, axis=0, keepdims=True)


def _mm_stats(x, w, b, mask, br=512):
    n, k = x.shape
    m = w.shape[1]
    return pl.pallas_call(
        _mm_stats_kernel,
        grid=(n // br,),
        in_specs=[
            pl.BlockSpec((br, k), lambda g: (g, 0)),
            pl.BlockSpec((k, m), lambda g: (0, 0)),
            pl.BlockSpec((1, m), lambda g: (0, 0)),
            pl.BlockSpec((br, 1), lambda g: (g, 0)),
        ],
        out_specs=[
            pl.BlockSpec((br, m), lambda g: (g, 0)),
            pl.BlockSpec((8, m), lambda g: (0, 0)),
        ],
        out_shape=[
            jax.ShapeDtypeStruct((n, m), _F32),
            jax.ShapeDtypeStruct((8, m), _F32),
        ],
    )(x, w, b.reshape(1, m), mask)


def _make_rbf_kernel(gamma):
    def k(d_ref, cen_ref, w_ref, b_ref, mask_ref, y_ref, st_ref):
        d = d_ref[...]
        f = jnp.exp(-gamma * (d - cen_ref[...]) ** 2)
        y = (jnp.dot(f, w_ref[...], preferred_element_type=_F32)
             + b_ref[...])
        y_ref[...] = y

        @pl.when(pl.program_id(0) == 0)
        def _():
            st_ref[...] = jnp.zeros(st_ref.shape, _F32)

        yv = y * mask_ref[...]
        st_ref[0:1, :] += jnp.sum(yv, axis=0, keepdims=True)
        st_ref[1:2, :] += jnp.sum(y * yv, axis=0, keepdims=True)

    return k


def _rbf_mm_stats(d, vmin, vmax, bins, w, b, mask, br=512):
    n = d.shape[0]
    m = w.shape[1]
    gamma = float((bins - 1) / (vmax - vmin))
    centers = jnp.linspace(vmin, vmax, bins, dtype=_F32).reshape(1, bins)
    return pl.pallas_call(
        _make_rbf_kernel(gamma),
        grid=(n // br,),
        in_specs=[
            pl.BlockSpec((br, 1), lambda g: (g, 0)),
            pl.BlockSpec((1, bins), lambda g: (0, 0)),
            pl.BlockSpec((bins, m), lambda g: (0, 0)),
            pl.BlockSpec((1, m), lambda g: (0, 0)),
            pl.BlockSpec((br, 1), lambda g: (g, 0)),
        ],
        out_specs=[
            pl.BlockSpec((br, m), lambda g: (g, 0)),
            pl.BlockSpec((8, m), lambda g: (0, 0)),
        ],
        out_shape=[
            jax.ShapeDtypeStruct((n, m), _F32),
            jax.ShapeDtypeStruct((8, m), _F32),
        ],
    )(d, centers, w, b.reshape(1, m), mask)


def _make_edge_kernel(R, B):
    def k(sp_ref, gs_ref, dt_ref, y_ref, w_ref, b_ref, dloc_ref, valid_ref,
          m_ref, nd_ref, st_ref):
        g = pl.program_id(0)
        a = gs_ref[:, :HID]
        c = gs_ref[:, HID:]
        eg = (jnp.dot(y_ref[...], w_ref[...], preferred_element_type=_F32)
              + b_ref[...])
        onehot = (dloc_ref[...]
                  == lax.broadcasted_iota(jnp.int32, (B, R), 1)).astype(_F32)
        b_rows = jnp.dot(onehot, dt_ref[...], preferred_element_type=_F32)
        m = a + b_rows + eg
        m_ref[...] = m
        valid = valid_ref[...]
        sig = jax.nn.sigmoid(m) * valid
        contrib = jnp.concatenate([sig * c, sig], axis=1)

        blk = sp_ref[g]
        prev = sp_ref[jnp.maximum(g - 1, 0)]

        @pl.when(jnp.logical_or(g == 0, blk != prev))
        def _():
            nd_ref[...] = jnp.zeros(nd_ref.shape, _F32)

        nd_ref[...] += lax.dot_general(
            onehot, contrib, (((0,), (0,)), ((), ())),
            preferred_element_type=_F32)

        @pl.when(g == 0)
        def _():
            st_ref[...] = jnp.zeros(st_ref.shape, _F32)

        mv = m * valid
        st_ref[0:1, :] += jnp.sum(mv, axis=0, keepdims=True)
        st_ref[1:2, :] += jnp.sum(m * mv, axis=0, keepdims=True)

    return k


def _edge_stage(lay, gs, dt, y, w, b, R, B, n_seg_pad):
    E_pad = gs.shape[0]
    G = E_pad // B
    grid_spec = pltpu.PrefetchScalarGridSpec(
        num_scalar_prefetch=1,
        grid=(G,),
        in_specs=[
            pl.BlockSpec((B, 2 * HID), lambda g, sp: (g, 0)),
            pl.BlockSpec((R, HID), lambda g, sp: (sp[g], 0)),
            pl.BlockSpec((B, HID), lambda g, sp: (g, 0)),
            pl.BlockSpec((HID, HID), lambda g, sp: (0, 0)),
            pl.BlockSpec((1, HID), lambda g, sp: (0, 0)),
            pl.BlockSpec((B, 1), lambda g, sp: (g, 0)),
            pl.BlockSpec((B, 1), lambda g, sp: (g, 0)),
        ],
        out_specs=[
            pl.BlockSpec((B, HID), lambda g, sp: (g, 0)),
            pl.BlockSpec((R, 2 * HID), lambda g, sp: (sp[g], 0)),
            pl.BlockSpec((8, HID), lambda g, sp: (0, 0)),
        ],
    )
    return pl.pallas_call(
        _make_edge_kernel(R, B),
        grid_spec=grid_spec,
        out_shape=[
            jax.ShapeDtypeStruct((E_pad, HID), _F32),
            jax.ShapeDtypeStruct((n_seg_pad, 2 * HID), _F32),
            jax.ShapeDtypeStruct((8, HID), _F32),
        ],
    )(lay["sp"], gs, dt, y, w, b.reshape(1, HID), lay["dloc"], lay["valid"])


def _node_kernel(x_ref, w_ref, b_ref, nd_ref, mask_ref, t_ref, st_ref):
    nd = nd_ref[...]
    h = nd[:, :HID] / (nd[:, HID:] + 1e-6)
    t = (jnp.dot(x_ref[...], w_ref[...], preferred_element_type=_F32)
         + b_ref[...] + h)
    t_ref[...] = t

    @pl.when(pl.program_id(0) == 0)
    def _():
        st_ref[...] = jnp.zeros(st_ref.shape, _F32)

    tv = t * mask_ref[...]
    st_ref[0:1, :] += jnp.sum(tv, axis=0, keepdims=True)
    st_ref[1:2, :] += jnp.sum(t * tv, axis=0, keepdims=True)


def _node_stage(x, w, b, nd, mask, br=512):
    n = x.shape[0]
    return pl.pallas_call(
        _node_kernel,
        grid=(n // br,),
        in_specs=[
            pl.BlockSpec((br, HID), lambda g: (g, 0)),
            pl.BlockSpec((HID, HID), lambda g: (0, 0)),
            pl.BlockSpec((1, HID), lambda g: (0, 0)),
            pl.BlockSpec((br, 2 * HID), lambda g: (g, 0)),
            pl.BlockSpec((br, 1), lambda g: (g, 0)),
        ],
        out_specs=[
            pl.BlockSpec((br, HID), lambda g: (g, 0)),
            pl.BlockSpec((8, HID), lambda g: (0, 0)),
        ],
        out_shape=[
            jax.ShapeDtypeStruct((n, HID), _F32),
            jax.ShapeDtypeStruct((8, HID), _F32),
        ],
    )(x, w, b.reshape(1, HID), nd, mask)


def _apply_res_kernel(y_ref, r_ref, sc_ref, sh_ref, o_ref):
    yb = y_ref[...] * sc_ref[...] + sh_ref[...]
    o_ref[...] = r_ref[...] + yb * jax.nn.sigmoid(yb)


def _apply_kernel(y_ref, sc_ref, sh_ref, o_ref):
    yb = y_ref[...] * sc_ref[...] + sh_ref[...]
    o_ref[...] = yb * jax.nn.sigmoid(yb)


def _apply(y, sc, sh, res=None, br=512):
    n, m = y.shape
    row = pl.BlockSpec((br, m), lambda g: (g, 0))
    one = pl.BlockSpec((1, m), lambda g: (0, 0))
    if res is None:
        return pl.pallas_call(
            _apply_kernel, grid=(n // br,),
            in_specs=[row, one, one], out_specs=row,
            out_shape=jax.ShapeDtypeStruct((n, m), _F32),
        )(y, sc.reshape(1, m), sh.reshape(1, m))
    return pl.pallas_call(
        _apply_res_kernel, grid=(n // br,),
        in_specs=[row, row, one, one], out_specs=row,
        out_shape=jax.ShapeDtypeStruct((n, m), _F32),
    )(y, res, sc.reshape(1, m), sh.reshape(1, m))


def _colsum_kernel(x_ref, mask_ref, st_ref):
    @pl.when(pl.program_id(0) == 0)
    def _():
        st_ref[...] = jnp.zeros(st_ref.shape, _F32)

    st_ref[0:1, :] += jnp.sum(x_ref[...] * mask_ref[...], axis=0,
                              keepdims=True)


def _colsum(x, mask, br=512):
    n, m = x.shape
    return pl.pallas_call(
        _colsum_kernel,
        grid=(n // br,),
        in_specs=[
            pl.BlockSpec((br, m), lambda g: (g, 0)),
            pl.BlockSpec((br, 1), lambda g: (g, 0)),
        ],
        out_specs=pl.BlockSpec((8, m), lambda g: (0, 0)),
        out_shape=jax.ShapeDtypeStruct((8, m), _F32),
    )(x, mask)


def _bn_affine(st, count, gamma, beta):
    s = st[0]
    ss = st[1]
    mu = s / count
    var = ss / count - mu * mu
    inv = gamma * lax.rsqrt(var + 1e-5)
    return inv, beta - mu * inv


def _mlp(p, x, mask, count):
    y, st = _mm_stats(x, p["w"], p["b"], mask)
    sc, sh = _bn_affine(st, count, p["g"], p["be"])
    return _apply(y, sc, sh)


def _eggc(p, lay, R, B, n_seg_pad, x, y, x_mask, x_count, y_count):
    wg = jnp.concatenate([p["src_gate_w"], p["dst_update_w"]], axis=1)
    bg = jnp.concatenate([p["src_gate_b"], p["dst_update_b"]])
    gt = _mm(x, wg, bg)
    dt = _mm(x, p["dst_gate_w"], p["dst_gate_b"])
    gs = _sc_gather(gt, lay["src"])
    m, nd, mst = _edge_stage(lay, gs, dt, y, p["edge_gate_w"],
                             p["edge_gate_b"], R, B, n_seg_pad)
    t, tst = _node_stage(x, p["src_update_w"], p["src_update_b"], nd, x_mask)
    tsc, tsh = _bn_affine(tst, x_count, p["bn_nodes_g"], p["bn_nodes_b"])
    msc, msh = _bn_affine(mst, y_count, p["bn_edges_g"], p["bn_edges_b"])
    x_out = _apply(t, tsc, tsh, res=x)
    y_out = _apply(m, msc, msh, res=y)
    return x_out, y_out


def kernel(atom_features, r, angle_h, edge_index, lg_edge_index, params):
    N = atom_features.shape[0]
    E = r.shape[0]
    T = angle_h.shape[0]
    R_G, B_G = 128, 128
    R_L, B_L = 128, 64

    N_pad = _round_up(N, 512)
    src = edge_index[0].astype(jnp.int32)
    dst = edge_index[1].astype(jnp.int32)
    gl = _build_layout(src, dst, N_pad, R_G, B_G)
    E_pad = gl["E_pad"]

    lsrc = jnp.take(gl["pos_of_orig"], lg_edge_index[0].astype(jnp.int32))
    ldst = jnp.take(gl["pos_of_orig"], lg_edge_index[1].astype(jnp.int32))
    ll = _build_layout(lsrc, ldst, E_pad, R_L, B_L)
    T_pad = ll["E_pad"]

    node_mask = (jnp.arange(N_pad) < N).astype(_F32).reshape(N_pad, 1)
    edge_mask = gl["valid"]
    ang_mask = ll["valid"]

    x0 = jnp.zeros((N_pad, atom_features.shape[1]), _F32).at[:N].set(
        atom_features)
    x = _mlp(params["atom_emb"], x0, node_mask, float(N))

    d_bond = jnp.sqrt(jnp.sum(r * r, axis=1))
    d_pad = jnp.take(d_bond, gl["take"]).reshape(E_pad, 1)
    p1 = params["edge_emb1"]
    y, st = _rbf_mm_stats(d_pad, 0.0, 8.0, 80, p1["w"], p1["b"], edge_mask)
    sc, sh = _bn_affine(st, float(E), p1["g"], p1["be"])
    y = _apply(y, sc, sh)
    y = _mlp(params["edge_emb2"], y, edge_mask, float(E))

    d_ang = jnp.take(angle_h, ll["take"]).reshape(T_pad, 1)
    p2 = params["angle_emb1"]
    z, st = _rbf_mm_stats(d_ang, -1.0, 1.0, 40, p2["w"], p2["b"], ang_mask)
    sc, sh = _bn_affine(st, float(T), p2["g"], p2["be"])
    z = _apply(z, sc, sh)
    z = _mlp(params["angle_emb2"], z, ang_mask, float(T))

    for lp in params["alignn"]:
        x, m = _eggc(lp["node"], gl, R_G, B_G, N_pad, x, y, node_mask,
                     float(N), float(E))
        y, z = _eggc(lp["edge"], ll, R_L, B_L, E_pad, m, z, edge_mask,
                     float(E), float(T))
    for lp in params["gcn"]:
        x, y = _eggc(lp, gl, R_G, B_G, N_pad, x, y, node_mask,
                     float(N), float(E))

    st = _colsum(x, node_mask)
    h = st[0] / float(N)
    out = h @ params["fc_w"] + params["fc_b"]
    return jnp.squeeze(out)


# speedup vs baseline: 1.6798x; 1.0603x over previous
import functools

import jax
import jax.numpy as jnp
from jax import lax
from jax.experimental import pallas as pl
from jax.experimental.pallas import tpu as pltpu
from jax.experimental.pallas import tpu_sc as plsc

_F32 = jnp.float32
HID = 256


def _round_up(x, m):
    return (x + m - 1) // m * m


def _build_layout(src, dst, n_seg_pad, R, B):
    E = dst.shape[0]
    K = n_seg_pad // R
    E_pad = _round_up(E + K * B, 8192)
    G = E_pad // B

    order = jnp.argsort(dst).astype(jnp.int32)
    dst_s = jnp.take(dst, order)
    blk = dst_s // R
    cnt = jnp.zeros((K,), jnp.int32).at[blk].add(1)
    pk = jnp.maximum(((cnt + B - 1) // B) * B, B)
    zero1 = jnp.zeros((1,), jnp.int32)
    pstart = jnp.concatenate([zero1, jnp.cumsum(pk)])[:K]
    first = jnp.concatenate([zero1, jnp.cumsum(cnt)])[:K]
    rank = jnp.arange(E, dtype=jnp.int32) - jnp.take(first, blk)
    pos = jnp.take(pstart, blk) + rank

    bstart = pstart // B
    sp = jnp.clip(
        jnp.searchsorted(bstart, jnp.arange(G, dtype=jnp.int32), side="right")
        .astype(jnp.int32) - 1, 0, K - 1)

    valid = jnp.zeros((E_pad, 1), _F32).at[pos, 0].set(1.0)
    dloc = jnp.zeros((E_pad, 1), jnp.int32).at[pos, 0].set(dst_s - blk * R)
    take = jnp.zeros((E_pad,), jnp.int32).at[pos].set(order)
    pos_of_orig = jnp.zeros((E,), jnp.int32).at[order].set(pos)
    return dict(
        E_pad=E_pad, G=G, K=K, sp=sp, valid=valid, dloc=dloc, take=take,
        pos_of_orig=pos_of_orig,
        src=jnp.take(src, take), dst=jnp.take(dst, take))


@functools.cache
def _make_gather(V, D, B_total):
    NW = 32
    b_per_w = B_total // NW
    C = max(c for c in (128, 112, 96, 80, 64, 48, 32, 16, 8)
            if b_per_w % c == 0 and (b_per_w // c) % 2 == 0
            and 2 * c * D * 4 <= 420 * 1024)
    nch = b_per_w // C
    mesh = plsc.VectorSubcoreMesh(core_axis_name="c", subcore_axis_name="s")

    @functools.partial(
        pl.kernel,
        out_type=jax.ShapeDtypeStruct((B_total, D), _F32),
        mesh=mesh,
        scratch_types=[
            pltpu.VMEM((C,), jnp.int32),
            pltpu.VMEM((C,), jnp.int32),
            pltpu.VMEM((C, D), _F32),
            pltpu.VMEM((C, D), _F32),
            pltpu.SemaphoreType.DMA,
            pltpu.SemaphoreType.DMA,
        ],
    )
    def gather(table_hbm, idx_hbm, out_hbm, idx0, idx1, rows0, rows1,
               sem0, sem1):
        wid = lax.axis_index("s") * 2 + lax.axis_index("c")
        base = wid * b_per_w
        idx_v = (idx0, idx1)
        rows_v = (rows0, rows1)
        sems = (sem0, sem1)

        pltpu.sync_copy(idx_hbm.at[pl.ds(base, C)], idx0)
        pltpu.async_copy(table_hbm.at[idx0], rows0, sem0)

        def body(k, carry):
            for b in range(2):
                j = 2 * k + b
                nb = 1 - b

                @pl.when(j + 1 < nch)
                def _(j=j, nb=nb):
                    off = base + (j + 1) * C
                    pltpu.sync_copy(idx_hbm.at[pl.ds(off, C)], idx_v[nb])
                    pltpu.async_copy(table_hbm.at[idx_v[nb]], rows_v[nb],
                                     sems[nb])

                pltpu.make_async_copy(table_hbm.at[idx_v[b]], rows_v[b],
                                      sems[b]).wait()
                pltpu.sync_copy(rows_v[b], out_hbm.at[pl.ds(base + j * C, C)])
            return carry

        lax.fori_loop(0, nch // 2, body, 0)

    return gather


def _sc_gather(table, idx):
    V, D = table.shape
    (Bt,) = idx.shape
    return _make_gather(V, D, Bt)(table, idx)


def _mm_kernel(x_ref, w_ref, b_ref, o_ref):
    o_ref[...] = (jnp.dot(x_ref[...], w_ref[...], preferred_element_type=_F32)
                  + b_ref[...])


def _mm_pack_kernel(x_ref, w_ref, b_ref, o_ref):
    xw = (jnp.dot(x_ref[...], w_ref[...], preferred_element_type=_F32)
          + b_ref[...])
    au = lax.bitcast_convert_type(xw[:, :HID], jnp.uint32)
    cu = lax.bitcast_convert_type(xw[:, HID:], jnp.uint32)
    packed = (au & jnp.uint32(0xFFFF0000)) | (cu >> jnp.uint32(16))
    o_ref[...] = lax.bitcast_convert_type(packed, _F32)


def _mm_pack(x, w, b, br=512):
    n, k = x.shape
    m = w.shape[1]
    return pl.pallas_call(
        _mm_pack_kernel,
        grid=(n // br,),
        in_specs=[
            pl.BlockSpec((br, k), lambda g: (g, 0)),
            pl.BlockSpec((k, m), lambda g: (0, 0)),
            pl.BlockSpec((1, m), lambda g: (0, 0)),
        ],
        out_specs=pl.BlockSpec((br, m // 2), lambda g: (g, 0)),
        out_shape=jax.ShapeDtypeStruct((n, m // 2), _F32),
    )(x, w, b.reshape(1, m))


def _mm(x, w, b, br=512):
    n, k = x.shape
    m = w.shape[1]
    return pl.pallas_call(
        _mm_kernel,
        grid=(n // br,),
        in_specs=[
            pl.BlockSpec((br, k), lambda g: (g, 0)),
            pl.BlockSpec((k, m), lambda g: (0, 0)),
            pl.BlockSpec((1, m), lambda g: (0, 0)),
        ],
        out_specs=pl.BlockSpec((br, m), lambda g: (g, 0)),
        out_shape=jax.ShapeDtypeStruct((n, m), _F32),
    )(x, w, b.reshape(1, m))


def _mm_stats_kernel(x_ref, w_ref, b_ref, mask_ref, y_ref, st_ref):
    y = (jnp.dot(x_ref[...], w_ref[...], preferred_element_type=_F32)
         + b_ref[...])
    y_ref[...] = y

    @pl.when(pl.program_id(0) == 0)
    def _():
        st_ref[...] = jnp.zeros(st_ref.shape, _F32)

    yv = y * mask_ref[...]
    st_ref[0:1, :] += jnp.sum(yv, axis=0, keepdims=True)
    st_ref[1:2, :] += jnp.sum(y * yv, axis=0, keepdims=True)


def _mm_stats(x, w, b, mask, br=512):
    n, k = x.shape
    m = w.shape[1]
    return pl.pallas_call(
        _mm_stats_kernel,
        grid=(n // br,),
        in_specs=[
            pl.BlockSpec((br, k), lambda g: (g, 0)),
            pl.BlockSpec((k, m), lambda g: (0, 0)),
            pl.BlockSpec((1, m), lambda g: (0, 0)),
            pl.BlockSpec((br, 1), lambda g: (g, 0)),
        ],
        out_specs=[
            pl.BlockSpec((br, m), lambda g: (g, 0)),
            pl.BlockSpec((8, m), lambda g: (0, 0)),
        ],
        out_shape=[
            jax.ShapeDtypeStruct((n, m), _F32),
            jax.ShapeDtypeStruct((8, m), _F32),
        ],
    )(x, w, b.reshape(1, m), mask)


def _make_rbf_kernel(gamma):
    def k(d_ref, cen_ref, w_ref, b_ref, mask_ref, y_ref, st_ref):
        d = d_ref[...]
        f = jnp.exp(-gamma * (d - cen_ref[...]) ** 2)
        y = (jnp.dot(f, w_ref[...], preferred_element_type=_F32)
             + b_ref[...])
        y_ref[...] = y

        @pl.when(pl.program_id(0) == 0)
        def _():
            st_ref[...] = jnp.zeros(st_ref.shape, _F32)

        yv = y * mask_ref[...]
        st_ref[0:1, :] += jnp.sum(yv, axis=0, keepdims=True)
        st_ref[1:2, :] += jnp.sum(y * yv, axis=0, keepdims=True)

    return k


def _rbf_mm_stats(d, vmin, vmax, bins, w, b, mask, br=512):
    n = d.shape[0]
    m = w.shape[1]
    gamma = float((bins - 1) / (vmax - vmin))
    centers = jnp.linspace(vmin, vmax, bins, dtype=_F32).reshape(1, bins)
    return pl.pallas_call(
        _make_rbf_kernel(gamma),
        grid=(n // br,),
        in_specs=[
            pl.BlockSpec((br, 1), lambda g: (g, 0)),
            pl.BlockSpec((1, bins), lambda g: (0, 0)),
            pl.BlockSpec((bins, m), lambda g: (0, 0)),
            pl.BlockSpec((1, m), lambda g: (0, 0)),
            pl.BlockSpec((br, 1), lambda g: (g, 0)),
        ],
        out_specs=[
            pl.BlockSpec((br, m), lambda g: (g, 0)),
            pl.BlockSpec((8, m), lambda g: (0, 0)),
        ],
        out_shape=[
            jax.ShapeDtypeStruct((n, m), _F32),
            jax.ShapeDtypeStruct((8, m), _F32),
        ],
    )(d, centers, w, b.reshape(1, m), mask)


def _make_edge_kernel(R, B):
    def k(sp_ref, gs_ref, dt_ref, y_ref, w_ref, b_ref, dloc_ref, valid_ref,
          m_ref, nd_ref, st_ref):
        g = pl.program_id(0)
        u = lax.bitcast_convert_type(gs_ref[...], jnp.uint32)
        a = lax.bitcast_convert_type(u & jnp.uint32(0xFFFF0000), _F32)
        c = lax.bitcast_convert_type(u << jnp.uint32(16), _F32)
        eg = (jnp.dot(y_ref[...], w_ref[...], preferred_element_type=_F32)
              + b_ref[...])
        onehot = (dloc_ref[...]
                  == lax.broadcasted_iota(jnp.int32, (B, R), 1)).astype(_F32)
        b_rows = jnp.dot(onehot, dt_ref[...], preferred_element_type=_F32)
        m = a + b_rows + eg
        m_ref[...] = m
        valid = valid_ref[...]
        sig = jax.nn.sigmoid(m) * valid
        contrib = jnp.concatenate([sig * c, sig], axis=1)

        blk = sp_ref[g]
        prev = sp_ref[jnp.maximum(g - 1, 0)]

        @pl.when(jnp.logical_or(g == 0, blk != prev))
        def _():
            nd_ref[...] = jnp.zeros(nd_ref.shape, _F32)

        nd_ref[...] += lax.dot_general(
            onehot, contrib, (((0,), (0,)), ((), ())),
            preferred_element_type=_F32)

        @pl.when(g == 0)
        def _():
            st_ref[...] = jnp.zeros(st_ref.shape, _F32)

        mv = m * valid
        st_ref[0:1, :] += jnp.sum(mv, axis=0, keepdims=True)
        st_ref[1:2, :] += jnp.sum(m * mv, axis=0, keepdims=True)

    return k


def _edge_stage(lay, gs, dt, y, w, b, R, B, n_seg_pad):
    E_pad = gs.shape[0]
    G = E_pad // B
    grid_spec = pltpu.PrefetchScalarGridSpec(
        num_scalar_prefetch=1,
        grid=(G,),
        in_specs=[
            pl.BlockSpec((B, HID), lambda g, sp: (g, 0)),
            pl.BlockSpec((R, HID), lambda g, sp: (sp[g], 0)),
            pl.BlockSpec((B, HID), lambda g, sp: (g, 0)),
            pl.BlockSpec((HID, HID), lambda g, sp: (0, 0)),
            pl.BlockSpec((1, HID), lambda g, sp: (0, 0)),
            pl.BlockSpec((B, 1), lambda g, sp: (g, 0)),
            pl.BlockSpec((B, 1), lambda g, sp: (g, 0)),
        ],
        out_specs=[
            pl.BlockSpec((B, HID), lambda g, sp: (g, 0)),
            pl.BlockSpec((R, 2 * HID), lambda g, sp: (sp[g], 0)),
            pl.BlockSpec((8, HID), lambda g, sp: (0, 0)),
        ],
    )
    return pl.pallas_call(
        _make_edge_kernel(R, B),
        grid_spec=grid_spec,
        out_shape=[
            jax.ShapeDtypeStruct((E_pad, HID), _F32),
            jax.ShapeDtypeStruct((n_seg_pad, 2 * HID), _F32),
            jax.ShapeDtypeStruct((8, HID), _F32),
        ],
    )(lay["sp"], gs, dt, y, w, b.reshape(1, HID), lay["dloc"], lay["valid"])


def _node_kernel(x_ref, w_ref, b_ref, nd_ref, mask_ref, t_ref, st_ref):
    nd = nd_ref[...]
    h = nd[:, :HID] / (nd[:, HID:] + 1e-6)
    t = (jnp.dot(x_ref[...], w_ref[...], preferred_element_type=_F32)
         + b_ref[...] + h)
    t_ref[...] = t

    @pl.when(pl.program_id(0) == 0)
    def _():
        st_ref[...] = jnp.zeros(st_ref.shape, _F32)

    tv = t * mask_ref[...]
    st_ref[0:1, :] += jnp.sum(tv, axis=0, keepdims=True)
    st_ref[1:2, :] += jnp.sum(t * tv, axis=0, keepdims=True)


def _node_stage(x, w, b, nd, mask, br=512):
    n = x.shape[0]
    return pl.pallas_call(
        _node_kernel,
        grid=(n // br,),
        in_specs=[
            pl.BlockSpec((br, HID), lambda g: (g, 0)),
            pl.BlockSpec((HID, HID), lambda g: (0, 0)),
            pl.BlockSpec((1, HID), lambda g: (0, 0)),
            pl.BlockSpec((br, 2 * HID), lambda g: (g, 0)),
            pl.BlockSpec((br, 1), lambda g: (g, 0)),
        ],
        out_specs=[
            pl.BlockSpec((br, HID), lambda g: (g, 0)),
            pl.BlockSpec((8, HID), lambda g: (0, 0)),
        ],
        out_shape=[
            jax.ShapeDtypeStruct((n, HID), _F32),
            jax.ShapeDtypeStruct((8, HID), _F32),
        ],
    )(x, w, b.reshape(1, HID), nd, mask)


def _apply_res_kernel(y_ref, r_ref, sc_ref, sh_ref, o_ref):
    yb = y_ref[...] * sc_ref[...] + sh_ref[...]
    o_ref[...] = r_ref[...] + yb * jax.nn.sigmoid(yb)


def _apply_kernel(y_ref, sc_ref, sh_ref, o_ref):
    yb = y_ref[...] * sc_ref[...] + sh_ref[...]
    o_ref[...] = yb * jax.nn.sigmoid(yb)


def _apply(y, sc, sh, res=None, br=512):
    n, m = y.shape
    row = pl.BlockSpec((br, m), lambda g: (g, 0))
    one = pl.BlockSpec((1, m), lambda g: (0, 0))
    if res is None:
        return pl.pallas_call(
            _apply_kernel, grid=(n // br,),
            in_specs=[row, one, one], out_specs=row,
            out_shape=jax.ShapeDtypeStruct((n, m), _F32),
        )(y, sc.reshape(1, m), sh.reshape(1, m))
    return pl.pallas_call(
        _apply_res_kernel, grid=(n // br,),
        in_specs=[row, row, one, one], out_specs=row,
        out_shape=jax.ShapeDtypeStruct((n, m), _F32),
    )(y, res, sc.reshape(1, m), sh.reshape(1, m))


def _colsum_kernel(x_ref, mask_ref, st_ref):
    @pl.when(pl.program_id(0) == 0)
    def _():
        st_ref[...] = jnp.zeros(st_ref.shape, _F32)

    st_ref[0:1, :] += jnp.sum(x_ref[...] * mask_ref[...], axis=0,
                              keepdims=True)


def _colsum(x, mask, br=512):
    n, m = x.shape
    return pl.pallas_call(
        _colsum_kernel,
        grid=(n // br,),
        in_specs=[
            pl.BlockSpec((br, m), lambda g: (g, 0)),
            pl.BlockSpec((br, 1), lambda g: (g, 0)),
        ],
        out_specs=pl.BlockSpec((8, m), lambda g: (0, 0)),
        out_shape=jax.ShapeDtypeStruct((8, m), _F32),
    )(x, mask)


def _bn_affine(st, count, gamma, beta):
    s = st[0]
    ss = st[1]
    mu = s / count
    var = ss / count - mu * mu
    inv = gamma * lax.rsqrt(var + 1e-5)
    return inv, beta - mu * inv


def _mlp(p, x, mask, count):
    y, st = _mm_stats(x, p["w"], p["b"], mask)
    sc, sh = _bn_affine(st, count, p["g"], p["be"])
    return _apply(y, sc, sh)


def _eggc(p, lay, R, B, n_seg_pad, x, y, x_mask, x_count, y_count):
    wg = jnp.concatenate([p["src_gate_w"], p["dst_update_w"]], axis=1)
    bg = jnp.concatenate([p["src_gate_b"], p["dst_update_b"]])
    gt = _mm_pack(x, wg, bg)
    dt = _mm(x, p["dst_gate_w"], p["dst_gate_b"])
    gs = _sc_gather(gt, lay["src"])
    m, nd, mst = _edge_stage(lay, gs, dt, y, p["edge_gate_w"],
                             p["edge_gate_b"], R, B, n_seg_pad)
    t, tst = _node_stage(x, p["src_update_w"], p["src_update_b"], nd, x_mask)
    tsc, tsh = _bn_affine(tst, x_count, p["bn_nodes_g"], p["bn_nodes_b"])
    msc, msh = _bn_affine(mst, y_count, p["bn_edges_g"], p["bn_edges_b"])
    x_out = _apply(t, tsc, tsh, res=x)
    y_out = _apply(m, msc, msh, res=y)
    return x_out, y_out


def kernel(atom_features, r, angle_h, edge_index, lg_edge_index, params):
    N = atom_features.shape[0]
    E = r.shape[0]
    T = angle_h.shape[0]
    R_G, B_G = 128, 128
    R_L, B_L = 128, 64

    N_pad = _round_up(N, 512)
    src = edge_index[0].astype(jnp.int32)
    dst = edge_index[1].astype(jnp.int32)
    gl = _build_layout(src, dst, N_pad, R_G, B_G)
    E_pad = gl["E_pad"]

    lsrc = jnp.take(gl["pos_of_orig"], lg_edge_index[0].astype(jnp.int32))
    ldst = jnp.take(gl["pos_of_orig"], lg_edge_index[1].astype(jnp.int32))
    ll = _build_layout(lsrc, ldst, E_pad, R_L, B_L)
    T_pad = ll["E_pad"]

    node_mask = (jnp.arange(N_pad) < N).astype(_F32).reshape(N_pad, 1)
    edge_mask = gl["valid"]
    ang_mask = ll["valid"]

    x0 = jnp.zeros((N_pad, atom_features.shape[1]), _F32).at[:N].set(
        atom_features)
    x = _mlp(params["atom_emb"], x0, node_mask, float(N))

    d_bond = jnp.sqrt(jnp.sum(r * r, axis=1))
    d_pad = jnp.take(d_bond, gl["take"]).reshape(E_pad, 1)
    p1 = params["edge_emb1"]
    y, st = _rbf_mm_stats(d_pad, 0.0, 8.0, 80, p1["w"], p1["b"], edge_mask)
    sc, sh = _bn_affine(st, float(E), p1["g"], p1["be"])
    y = _apply(y, sc, sh)
    y = _mlp(params["edge_emb2"], y, edge_mask, float(E))

    d_ang = jnp.take(angle_h, ll["take"]).reshape(T_pad, 1)
    p2 = params["angle_emb1"]
    z, st = _rbf_mm_stats(d_ang, -1.0, 1.0, 40, p2["w"], p2["b"], ang_mask)
    sc, sh = _bn_affine(st, float(T), p2["g"], p2["be"])
    z = _apply(z, sc, sh)
    z = _mlp(params["angle_emb2"], z, ang_mask, float(T))

    for lp in params["alignn"]:
        x, m = _eggc(lp["node"], gl, R_G, B_G, N_pad, x, y, node_mask,
                     float(N), float(E))
        y, z = _eggc(lp["edge"], ll, R_L, B_L, E_pad, m, z, edge_mask,
                     float(E), float(T))
    for lp in params["gcn"]:
        x, y = _eggc(lp, gl, R_G, B_G, N_pad, x, y, node_mask,
                     float(N), float(E))

    st = _colsum(x, node_mask)
    h = st[0] / float(N)
    out = h @ params["fc_w"] + params["fc_b"]
    return jnp.squeeze(out)


# speedup vs baseline: 1.6848x; 1.0030x over previous
import functools

import jax
import jax.numpy as jnp
from jax import lax
from jax.experimental import pallas as pl
from jax.experimental.pallas import tpu as pltpu
from jax.experimental.pallas import tpu_sc as plsc

_F32 = jnp.float32
_BF16 = jnp.bfloat16
HID = 256


def _dotf(a, b):
    return jnp.dot(a.astype(_BF16), b.astype(_BF16),
                   preferred_element_type=_F32)


def _round_up(x, m):
    return (x + m - 1) // m * m


def _build_layout(src, dst, n_seg_pad, R, B):
    E = dst.shape[0]
    K = n_seg_pad // R
    E_pad = _round_up(E + K * B, 8192)
    G = E_pad // B

    order = jnp.argsort(dst).astype(jnp.int32)
    dst_s = jnp.take(dst, order)
    blk = dst_s // R
    cnt = jnp.zeros((K,), jnp.int32).at[blk].add(1)
    pk = jnp.maximum(((cnt + B - 1) // B) * B, B)
    zero1 = jnp.zeros((1,), jnp.int32)
    pstart = jnp.concatenate([zero1, jnp.cumsum(pk)])[:K]
    first = jnp.concatenate([zero1, jnp.cumsum(cnt)])[:K]
    rank = jnp.arange(E, dtype=jnp.int32) - jnp.take(first, blk)
    pos = jnp.take(pstart, blk) + rank

    bstart = pstart // B
    sp = jnp.clip(
        jnp.searchsorted(bstart, jnp.arange(G, dtype=jnp.int32), side="right")
        .astype(jnp.int32) - 1, 0, K - 1)

    valid = jnp.zeros((E_pad, 1), _F32).at[pos, 0].set(1.0)
    dloc = jnp.zeros((E_pad, 1), jnp.int32).at[pos, 0].set(dst_s - blk * R)
    take = jnp.zeros((E_pad,), jnp.int32).at[pos].set(order)
    pos_of_orig = jnp.zeros((E,), jnp.int32).at[order].set(pos)
    return dict(
        E_pad=E_pad, G=G, K=K, sp=sp, valid=valid, dloc=dloc, take=take,
        pos_of_orig=pos_of_orig,
        src=jnp.take(src, take), dst=jnp.take(dst, take))


@functools.cache
def _make_gather(V, D, B_total):
    NW = 32
    b_per_w = B_total // NW
    C = max(c for c in (128, 112, 96, 80, 64, 48, 32, 16, 8)
            if b_per_w % c == 0 and (b_per_w // c) % 2 == 0
            and 2 * c * D * 4 <= 420 * 1024)
    nch = b_per_w // C
    mesh = plsc.VectorSubcoreMesh(core_axis_name="c", subcore_axis_name="s")

    @functools.partial(
        pl.kernel,
        out_type=jax.ShapeDtypeStruct((B_total, D), _F32),
        mesh=mesh,
        scratch_types=[
            pltpu.VMEM((C,), jnp.int32),
            pltpu.VMEM((C,), jnp.int32),
            pltpu.VMEM((C, D), _F32),
            pltpu.VMEM((C, D), _F32),
            pltpu.SemaphoreType.DMA,
            pltpu.SemaphoreType.DMA,
        ],
    )
    def gather(table_hbm, idx_hbm, out_hbm, idx0, idx1, rows0, rows1,
               sem0, sem1):
        wid = lax.axis_index("s") * 2 + lax.axis_index("c")
        base = wid * b_per_w
        idx_v = (idx0, idx1)
        rows_v = (rows0, rows1)
        sems = (sem0, sem1)

        pltpu.sync_copy(idx_hbm.at[pl.ds(base, C)], idx0)
        pltpu.async_copy(table_hbm.at[idx0], rows0, sem0)

        def body(k, carry):
            for b in range(2):
                j = 2 * k + b
                nb = 1 - b

                @pl.when(j + 1 < nch)
                def _(j=j, nb=nb):
                    off = base + (j + 1) * C
                    pltpu.sync_copy(idx_hbm.at[pl.ds(off, C)], idx_v[nb])
                    pltpu.async_copy(table_hbm.at[idx_v[nb]], rows_v[nb],
                                     sems[nb])

                pltpu.make_async_copy(table_hbm.at[idx_v[b]], rows_v[b],
                                      sems[b]).wait()
                pltpu.sync_copy(rows_v[b], out_hbm.at[pl.ds(base + j * C, C)])
            return carry

        lax.fori_loop(0, nch // 2, body, 0)

    return gather


def _sc_gather(table, idx):
    V, D = table.shape
    (Bt,) = idx.shape
    return _make_gather(V, D, Bt)(table, idx)


def _mm_kernel(x_ref, w_ref, b_ref, o_ref):
    o_ref[...] = _dotf(x_ref[...], w_ref[...]) + b_ref[...]


def _mm_pack_kernel(x_ref, w_ref, b_ref, o_ref):
    xw = _dotf(x_ref[...], w_ref[...]) + b_ref[...]
    au = lax.bitcast_convert_type(xw[:, :HID], jnp.uint32)
    cu = lax.bitcast_convert_type(xw[:, HID:], jnp.uint32)
    packed = (au & jnp.uint32(0xFFFF0000)) | (cu >> jnp.uint32(16))
    o_ref[...] = lax.bitcast_convert_type(packed, _F32)


def _mm_pack(x, w, b, br=512):
    n, k = x.shape
    m = w.shape[1]
    return pl.pallas_call(
        _mm_pack_kernel,
        grid=(n // br,),
        in_specs=[
            pl.BlockSpec((br, k), lambda g: (g, 0)),
            pl.BlockSpec((k, m), lambda g: (0, 0)),
            pl.BlockSpec((1, m), lambda g: (0, 0)),
        ],
        out_specs=pl.BlockSpec((br, m // 2), lambda g: (g, 0)),
        out_shape=jax.ShapeDtypeStruct((n, m // 2), _F32),
    )(x, w, b.reshape(1, m))


def _mm(x, w, b, br=512):
    n, k = x.shape
    m = w.shape[1]
    return pl.pallas_call(
        _mm_kernel,
        grid=(n // br,),
        in_specs=[
            pl.BlockSpec((br, k), lambda g: (g, 0)),
            pl.BlockSpec((k, m), lambda g: (0, 0)),
            pl.BlockSpec((1, m), lambda g: (0, 0)),
        ],
        out_specs=pl.BlockSpec((br, m), lambda g: (g, 0)),
        out_shape=jax.ShapeDtypeStruct((n, m), _F32),
    )(x, w, b.reshape(1, m))


def _mm_stats_kernel(x_ref, w_ref, b_ref, mask_ref, y_ref, st_ref):
    y = _dotf(x_ref[...], w_ref[...]) + b_ref[...]
    y_ref[...] = y

    @pl.when(pl.program_id(0) == 0)
    def _():
        st_ref[...] = jnp.zeros(st_ref.shape, _F32)

    yv = y * mask_ref[...]
    st_ref[0:1, :] += jnp.sum(yv, axis=0, keepdims=True)
    st_ref[1:2, :] += jnp.sum(y * yv, axis=0, keepdims=True)


def _mm_stats(x, w, b, mask, br=512):
    n, k = x.shape
    m = w.shape[1]
    return pl.pallas_call(
        _mm_stats_kernel,
        grid=(n // br,),
        in_specs=[
            pl.BlockSpec((br, k), lambda g: (g, 0)),
            pl.BlockSpec((k, m), lambda g: (0, 0)),
            pl.BlockSpec((1, m), lambda g: (0, 0)),
            pl.BlockSpec((br, 1), lambda g: (g, 0)),
        ],
        out_specs=[
            pl.BlockSpec((br, m), lambda g: (g, 0)),
            pl.BlockSpec((8, m), lambda g: (0, 0)),
        ],
        out_shape=[
            jax.ShapeDtypeStruct((n, m), _F32),
            jax.ShapeDtypeStruct((8, m), _F32),
        ],
    )(x, w, b.reshape(1, m), mask)


def _make_rbf_kernel(gamma):
    def k(d_ref, cen_ref, w_ref, b_ref, mask_ref, y_ref, st_ref):
        d = d_ref[...]
        f = jnp.exp(-gamma * (d - cen_ref[...]) ** 2)
        y = _dotf(f, w_ref[...]) + b_ref[...]
        y_ref[...] = y

        @pl.when(pl.program_id(0) == 0)
        def _():
            st_ref[...] = jnp.zeros(st_ref.shape, _F32)

        yv = y * mask_ref[...]
        st_ref[0:1, :] += jnp.sum(yv, axis=0, keepdims=True)
        st_ref[1:2, :] += jnp.sum(y * yv, axis=0, keepdims=True)

    return k


def _rbf_mm_stats(d, vmin, vmax, bins, w, b, mask, br=512):
    n = d.shape[0]
    m = w.shape[1]
    gamma = float((bins - 1) / (vmax - vmin))
    centers = jnp.linspace(vmin, vmax, bins, dtype=_F32).reshape(1, bins)
    return pl.pallas_call(
        _make_rbf_kernel(gamma),
        grid=(n // br,),
        in_specs=[
            pl.BlockSpec((br, 1), lambda g: (g, 0)),
            pl.BlockSpec((1, bins), lambda g: (0, 0)),
            pl.BlockSpec((bins, m), lambda g: (0, 0)),
            pl.BlockSpec((1, m), lambda g: (0, 0)),
            pl.BlockSpec((br, 1), lambda g: (g, 0)),
        ],
        out_specs=[
            pl.BlockSpec((br, m), lambda g: (g, 0)),
            pl.BlockSpec((8, m), lambda g: (0, 0)),
        ],
        out_shape=[
            jax.ShapeDtypeStruct((n, m), _F32),
            jax.ShapeDtypeStruct((8, m), _F32),
        ],
    )(d, centers, w, b.reshape(1, m), mask)


def _make_edge_kernel(R, B):
    def k(sp_ref, gs_ref, dt_ref, y_ref, w_ref, b_ref, dloc_ref, valid_ref,
          m_ref, nd_ref, st_ref):
        g = pl.program_id(0)
        u = lax.bitcast_convert_type(gs_ref[...], jnp.uint32)
        a = lax.bitcast_convert_type(u & jnp.uint32(0xFFFF0000), _F32)
        c = lax.bitcast_convert_type(u << jnp.uint32(16), _F32)
        eg = _dotf(y_ref[...], w_ref[...]) + b_ref[...]
        onehot = (dloc_ref[...]
                  == lax.broadcasted_iota(jnp.int32, (B, R), 1)).astype(_F32)
        b_rows = _dotf(onehot, dt_ref[...])
        m = a + b_rows + eg
        m_ref[...] = m
        valid = valid_ref[...]
        sig = jax.nn.sigmoid(m) * valid
        contrib = jnp.concatenate([sig * c, sig], axis=1)

        blk = sp_ref[g]
        prev = sp_ref[jnp.maximum(g - 1, 0)]

        @pl.when(jnp.logical_or(g == 0, blk != prev))
        def _():
            nd_ref[...] = jnp.zeros(nd_ref.shape, _F32)

        nd_ref[...] += lax.dot_general(
            onehot.astype(_BF16), contrib.astype(_BF16),
            (((0,), (0,)), ((), ())), preferred_element_type=_F32)

        @pl.when(g == 0)
        def _():
            st_ref[...] = jnp.zeros(st_ref.shape, _F32)

        mv = m * valid
        st_ref[0:1, :] += jnp.sum(mv, axis=0, keepdims=True)
        st_ref[1:2, :] += jnp.sum(m * mv, axis=0, keepdims=True)

    return k


def _edge_stage(lay, gs, dt, y, w, b, R, B, n_seg_pad):
    E_pad = gs.shape[0]
    G = E_pad // B
    grid_spec = pltpu.PrefetchScalarGridSpec(
        num_scalar_prefetch=1,
        grid=(G,),
        in_specs=[
            pl.BlockSpec((B, HID), lambda g, sp: (g, 0)),
            pl.BlockSpec((R, HID), lambda g, sp: (sp[g], 0)),
            pl.BlockSpec((B, HID), lambda g, sp: (g, 0)),
            pl.BlockSpec((HID, HID), lambda g, sp: (0, 0)),
            pl.BlockSpec((1, HID), lambda g, sp: (0, 0)),
            pl.BlockSpec((B, 1), lambda g, sp: (g, 0)),
            pl.BlockSpec((B, 1), lambda g, sp: (g, 0)),
        ],
        out_specs=[
            pl.BlockSpec((B, HID), lambda g, sp: (g, 0)),
            pl.BlockSpec((R, 2 * HID), lambda g, sp: (sp[g], 0)),
            pl.BlockSpec((8, HID), lambda g, sp: (0, 0)),
        ],
    )
    return pl.pallas_call(
        _make_edge_kernel(R, B),
        grid_spec=grid_spec,
        out_shape=[
            jax.ShapeDtypeStruct((E_pad, HID), _F32),
            jax.ShapeDtypeStruct((n_seg_pad, 2 * HID), _F32),
            jax.ShapeDtypeStruct((8, HID), _F32),
        ],
    )(lay["sp"], gs, dt, y, w, b.reshape(1, HID), lay["dloc"], lay["valid"])


def _node_kernel(x_ref, w_ref, b_ref, nd_ref, mask_ref, t_ref, st_ref):
    nd = nd_ref[...]
    h = nd[:, :HID] / (nd[:, HID:] + 1e-6)
    t = _dotf(x_ref[...], w_ref[...]) + b_ref[...] + h
    t_ref[...] = t

    @pl.when(pl.program_id(0) == 0)
    def _():
        st_ref[...] = jnp.zeros(st_ref.shape, _F32)

    tv = t * mask_ref[...]
    st_ref[0:1, :] += jnp.sum(tv, axis=0, keepdims=True)
    st_ref[1:2, :] += jnp.sum(t * tv, axis=0, keepdims=True)


def _node_stage(x, w, b, nd, mask, br=512):
    n = x.shape[0]
    return pl.pallas_call(
        _node_kernel,
        grid=(n // br,),
        in_specs=[
            pl.BlockSpec((br, HID), lambda g: (g, 0)),
            pl.BlockSpec((HID, HID), lambda g: (0, 0)),
            pl.BlockSpec((1, HID), lambda g: (0, 0)),
            pl.BlockSpec((br, 2 * HID), lambda g: (g, 0)),
            pl.BlockSpec((br, 1), lambda g: (g, 0)),
        ],
        out_specs=[
            pl.BlockSpec((br, HID), lambda g: (g, 0)),
            pl.BlockSpec((8, HID), lambda g: (0, 0)),
        ],
        out_shape=[
            jax.ShapeDtypeStruct((n, HID), _F32),
            jax.ShapeDtypeStruct((8, HID), _F32),
        ],
    )(x, w, b.reshape(1, HID), nd, mask)


def _apply_res_kernel(y_ref, r_ref, sc_ref, sh_ref, o_ref):
    yb = y_ref[...] * sc_ref[...] + sh_ref[...]
    o_ref[...] = r_ref[...] + yb * jax.nn.sigmoid(yb)


def _apply_kernel(y_ref, sc_ref, sh_ref, o_ref):
    yb = y_ref[...] * sc_ref[...] + sh_ref[...]
    o_ref[...] = yb * jax.nn.sigmoid(yb)


def _apply(y, sc, sh, res=None, br=512):
    n, m = y.shape
    row = pl.BlockSpec((br, m), lambda g: (g, 0))
    one = pl.BlockSpec((1, m), lambda g: (0, 0))
    if res is None:
        return pl.pallas_call(
            _apply_kernel, grid=(n // br,),
            in_specs=[row, one, one], out_specs=row,
            out_shape=jax.ShapeDtypeStruct((n, m), _F32),
        )(y, sc.reshape(1, m), sh.reshape(1, m))
    return pl.pallas_call(
        _apply_res_kernel, grid=(n // br,),
        in_specs=[row, row, one, one], out_specs=row,
        out_shape=jax.ShapeDtypeStruct((n, m), _F32),
    )(y, res, sc.reshape(1, m), sh.reshape(1, m))


def _colsum_kernel(x_ref, mask_ref, st_ref):
    @pl.when(pl.program_id(0) == 0)
    def _():
        st_ref[...] = jnp.zeros(st_ref.shape, _F32)

    st_ref[0:1, :] += jnp.sum(x_ref[...] * mask_ref[...], axis=0,
                              keepdims=True)


def _colsum(x, mask, br=512):
    n, m = x.shape
    return pl.pallas_call(
        _colsum_kernel,
        grid=(n // br,),
        in_specs=[
            pl.BlockSpec((br, m), lambda g: (g, 0)),
            pl.BlockSpec((br, 1), lambda g: (g, 0)),
        ],
        out_specs=pl.BlockSpec((8, m), lambda g: (0, 0)),
        out_shape=jax.ShapeDtypeStruct((8, m), _F32),
    )(x, mask)


def _bn_affine(st, count, gamma, beta):
    s = st[0]
    ss = st[1]
    mu = s / count
    var = ss / count - mu * mu
    inv = gamma * lax.rsqrt(var + 1e-5)
    return inv, beta - mu * inv


def _mlp(p, x, mask, count):
    y, st = _mm_stats(x, p["w"], p["b"], mask)
    sc, sh = _bn_affine(st, count, p["g"], p["be"])
    return _apply(y, sc, sh)


def _eggc(p, lay, R, B, n_seg_pad, x, y, x_mask, x_count, y_count):
    wg = jnp.concatenate([p["src_gate_w"], p["dst_update_w"]], axis=1)
    bg = jnp.concatenate([p["src_gate_b"], p["dst_update_b"]])
    gt = _mm_pack(x, wg, bg)
    dt = _mm(x, p["dst_gate_w"], p["dst_gate_b"])
    gs = _sc_gather(gt, lay["src"])
    m, nd, mst = _edge_stage(lay, gs, dt, y, p["edge_gate_w"],
                             p["edge_gate_b"], R, B, n_seg_pad)
    t, tst = _node_stage(x, p["src_update_w"], p["src_update_b"], nd, x_mask)
    tsc, tsh = _bn_affine(tst, x_count, p["bn_nodes_g"], p["bn_nodes_b"])
    msc, msh = _bn_affine(mst, y_count, p["bn_edges_g"], p["bn_edges_b"])
    x_out = _apply(t, tsc, tsh, res=x)
    y_out = _apply(m, msc, msh, res=y)
    return x_out, y_out


def kernel(atom_features, r, angle_h, edge_index, lg_edge_index, params):
    N = atom_features.shape[0]
    E = r.shape[0]
    T = angle_h.shape[0]
    R_G, B_G = 128, 128
    R_L, B_L = 128, 64

    N_pad = _round_up(N, 512)
    src = edge_index[0].astype(jnp.int32)
    dst = edge_index[1].astype(jnp.int32)
    gl = _build_layout(src, dst, N_pad, R_G, B_G)
    E_pad = gl["E_pad"]

    lsrc = jnp.take(gl["pos_of_orig"], lg_edge_index[0].astype(jnp.int32))
    ldst = jnp.take(gl["pos_of_orig"], lg_edge_index[1].astype(jnp.int32))
    ll = _build_layout(lsrc, ldst, E_pad, R_L, B_L)
    T_pad = ll["E_pad"]

    node_mask = (jnp.arange(N_pad) < N).astype(_F32).reshape(N_pad, 1)
    edge_mask = gl["valid"]
    ang_mask = ll["valid"]

    x0 = jnp.zeros((N_pad, atom_features.shape[1]), _F32).at[:N].set(
        atom_features)
    x = _mlp(params["atom_emb"], x0, node_mask, float(N))

    d_bond = jnp.sqrt(jnp.sum(r * r, axis=1))
    d_pad = jnp.take(d_bond, gl["take"]).reshape(E_pad, 1)
    p1 = params["edge_emb1"]
    y, st = _rbf_mm_stats(d_pad, 0.0, 8.0, 80, p1["w"], p1["b"], edge_mask)
    sc, sh = _bn_affine(st, float(E), p1["g"], p1["be"])
    y = _apply(y, sc, sh)
    y = _mlp(params["edge_emb2"], y, edge_mask, float(E))

    d_ang = jnp.take(angle_h, ll["take"]).reshape(T_pad, 1)
    p2 = params["angle_emb1"]
    z, st = _rbf_mm_stats(d_ang, -1.0, 1.0, 40, p2["w"], p2["b"], ang_mask)
    sc, sh = _bn_affine(st, float(T), p2["g"], p2["be"])
    z = _apply(z, sc, sh)
    z = _mlp(params["angle_emb2"], z, ang_mask, float(T))

    for lp in params["alignn"]:
        x, m = _eggc(lp["node"], gl, R_G, B_G, N_pad, x, y, node_mask,
                     float(N), float(E))
        y, z = _eggc(lp["edge"], ll, R_L, B_L, E_pad, m, z, edge_mask,
                     float(E), float(T))
    for lp in params["gcn"]:
        x, y = _eggc(lp, gl, R_G, B_G, N_pad, x, y, node_mask,
                     float(N), float(E))

    st = _colsum(x, node_mask)
    h = st[0] / float(N)
    out = h @ params["fc_w"] + params["fc_b"]
    return jnp.squeeze(out)
